# baseline (speedup 1.0000x reference)
import functools

import jax
import jax.numpy as jnp
from jax.experimental import pallas as pl
from jax.experimental.pallas import tpu as pltpu


def _gather_kernel(labels_ref, emb_ref, out_ref):
    n = labels_ref.shape[0]
    lpad = emb_ref.shape[0]
    lab = labels_ref[:]
    iota = jax.lax.broadcasted_iota(jnp.int32, (n, lpad), 1)
    onehot = (lab == iota).astype(jnp.float32)
    out_ref[:] = jnp.dot(onehot, emb_ref[:], preferred_element_type=jnp.float32)


def _embedding_gather(node_labels, emb):
    n = node_labels.shape[0]
    l, d = emb.shape
    lpad = ((l + 127) // 128) * 128
    emb_p = jnp.pad(emb, ((0, lpad - l), (0, 0)))
    labels2d = node_labels.astype(jnp.int32).reshape(n, 1)
    return pl.pallas_call(
        _gather_kernel,
        out_shape=jax.ShapeDtypeStruct((n, d), jnp.float32),
    )(labels2d, emb_p)


def _step_kernel(adj_ref, h_ref, We_ref, be_ref, Wz_ref, Wr_ref, Wh_ref,
                 bz_ref, br_ref, bh_ref, out_ref,
                 msgs_ref, acc_in_ref, acc_out_ref, *, t_fwd, bm):
    i = pl.program_id(0)
    t = pl.program_id(1)
    nt = pl.num_programs(1)
    d = h_ref.shape[1]

    @pl.when(i == 0)
    def _():
        msgs_ref[t] = (
            jnp.dot(h_ref[:], We_ref[t], preferred_element_type=jnp.float32)
            + be_ref[pl.ds(t, 1), :]
        )

    p = jnp.dot(adj_ref[0], msgs_ref[t], preferred_element_type=jnp.float32)

    @pl.when(t == 0)
    def _():
        acc_in_ref[:] = p

    @pl.when(jnp.logical_and(t > 0, t < t_fwd))
    def _():
        acc_in_ref[:] += p

    @pl.when(t == t_fwd)
    def _():
        acc_out_ref[:] = p

    @pl.when(t > t_fwd)
    def _():
        acc_out_ref[:] += p

    @pl.when(t == nt - 1)
    def _():
        h_blk = h_ref[pl.ds(i * bm, bm), :]
        a_in = acc_in_ref[:]
        a_out = acc_out_ref[:]

        def lin3(a, b, c, w_ref, bias_ref):
            return (
                jnp.dot(a, w_ref[0:d, :], preferred_element_type=jnp.float32)
                + jnp.dot(b, w_ref[d:2 * d, :], preferred_element_type=jnp.float32)
                + jnp.dot(c, w_ref[2 * d:3 * d, :], preferred_element_type=jnp.float32)
                + bias_ref[:]
            )

        z = jax.nn.sigmoid(lin3(a_in, a_out, h_blk, Wz_ref, bz_ref))
        r = jax.nn.sigmoid(lin3(a_in, a_out, h_blk, Wr_ref, br_ref))
        h_hat = jnp.tanh(lin3(a_in, a_out, r * h_blk, Wh_ref, bh_ref))
        out_ref[:] = (1.0 - z) * h_blk + z * h_hat


def _prop_step(adj_tensor, h, W_edge, b_edge, Wz, bz, Wr, br, Wh, bh,
               *, bm, interpret=False):
    nt, n, _ = adj_tensor.shape
    d = h.shape[1]
    nb = n // bm
    t_fwd = nt // 2

    full = lambda shape: pl.BlockSpec(shape, lambda i, t: (0,) * len(shape))
    grid_spec = pltpu.PrefetchScalarGridSpec(
        num_scalar_prefetch=0,
        grid=(nb, nt),
        in_specs=[
            pl.BlockSpec((1, bm, n), lambda i, t: (t, i, 0)),
            full((n, d)),
            full((nt, d, d)),
            full((nt, d)),
            full((3 * d, d)),
            full((3 * d, d)),
            full((3 * d, d)),
            full((1, d)),
            full((1, d)),
            full((1, d)),
        ],
        out_specs=pl.BlockSpec((bm, d), lambda i, t: (i, 0)),
        scratch_shapes=[
            pltpu.VMEM((nt, n, d), jnp.float32),
            pltpu.VMEM((bm, d), jnp.float32),
            pltpu.VMEM((bm, d), jnp.float32),
        ],
    )
    return pl.pallas_call(
        functools.partial(_step_kernel, t_fwd=t_fwd, bm=bm),
        grid_spec=grid_spec,
        out_shape=jax.ShapeDtypeStruct((n, d), jnp.float32),
        compiler_params=pltpu.CompilerParams(
            dimension_semantics=("arbitrary", "arbitrary"),
        ),
        interpret=interpret,
    )(adj_tensor, h, W_edge, b_edge, Wz, Wr, Wh,
      bz.reshape(1, d), br.reshape(1, d), bh.reshape(1, d))


def kernel(adj_tensor, node_labels, n_prop_steps, emb, W_edge, b_edge,
           Wz, bz, Wr, br, Wh, bh):
    h0 = _embedding_gather(node_labels, emb)

    def body(_, h):
        return _prop_step(adj_tensor, h, W_edge, b_edge, Wz, bz, Wr, br,
                          Wh, bh, bm=256)

    return jax.lax.fori_loop(0, n_prop_steps, body, h0)


# speedup vs baseline: 1.0509x; 1.0509x over previous
import functools

import jax
import jax.numpy as jnp
from jax.experimental import pallas as pl
from jax.experimental.pallas import tpu as pltpu


def _gather_kernel(labels_ref, emb_ref, out_ref):
    n = labels_ref.shape[0]
    lpad = emb_ref.shape[0]
    lab = labels_ref[:]
    iota = jax.lax.broadcasted_iota(jnp.int32, (n, lpad), 1)
    onehot = (lab == iota).astype(jnp.float32)
    out_ref[:] = jnp.dot(onehot, emb_ref[:], preferred_element_type=jnp.float32)


def _embedding_gather(node_labels, emb):
    n = node_labels.shape[0]
    l, d = emb.shape
    lpad = ((l + 127) // 128) * 128
    emb_p = jnp.pad(emb, ((0, lpad - l), (0, 0)))
    labels2d = node_labels.astype(jnp.int32).reshape(n, 1)
    return pl.pallas_call(
        _gather_kernel,
        out_shape=jax.ShapeDtypeStruct((n, d), jnp.float32),
    )(labels2d, emb_p)


def _step_kernel(adj_ref, h_ref, We_ref, be_ref, Wz_ref, Wr_ref, Wh_ref,
                 bz_ref, br_ref, bh_ref, out_ref, adj16_ref,
                 msgs_ref, acc_in_ref, acc_out_ref, *, t_fwd, bm, write_bf16):
    i = pl.program_id(0)
    t = pl.program_id(1)
    nt = pl.num_programs(1)
    d = h_ref.shape[1]

    @pl.when(i == 0)
    def _():
        m = (jnp.dot(h_ref[:], We_ref[t], preferred_element_type=jnp.float32)
             + be_ref[pl.ds(t, 1), :])
        msgs_ref[t] = m.astype(jnp.bfloat16)

    adj_blk = adj_ref[0]
    if adj_blk.dtype != jnp.bfloat16:
        adj_blk = adj_blk.astype(jnp.bfloat16)
    if write_bf16:
        adj16_ref[0] = adj_blk

    p = jnp.dot(adj_blk, msgs_ref[t], preferred_element_type=jnp.float32)

    @pl.when(t == 0)
    def _():
        acc_in_ref[:] = p

    @pl.when(jnp.logical_and(t > 0, t < t_fwd))
    def _():
        acc_in_ref[:] += p

    @pl.when(t == t_fwd)
    def _():
        acc_out_ref[:] = p

    @pl.when(t > t_fwd)
    def _():
        acc_out_ref[:] += p

    @pl.when(t == nt - 1)
    def _():
        h_blk = h_ref[pl.ds(i * bm, bm), :]
        a_in = acc_in_ref[:]
        a_out = acc_out_ref[:]

        def lin3(a, b, c, w_ref, bias_ref):
            return (
                jnp.dot(a, w_ref[0:d, :], preferred_element_type=jnp.float32)
                + jnp.dot(b, w_ref[d:2 * d, :], preferred_element_type=jnp.float32)
                + jnp.dot(c, w_ref[2 * d:3 * d, :], preferred_element_type=jnp.float32)
                + bias_ref[:]
            )

        z = jax.nn.sigmoid(lin3(a_in, a_out, h_blk, Wz_ref, bz_ref))
        r = jax.nn.sigmoid(lin3(a_in, a_out, h_blk, Wr_ref, br_ref))
        h_hat = jnp.tanh(lin3(a_in, a_out, r * h_blk, Wh_ref, bh_ref))
        out_ref[:] = (1.0 - z) * h_blk + z * h_hat


def _prop_step(adj_tensor, h, W_edge, b_edge, Wz, bz, Wr, br, Wh, bh,
               *, bm, write_bf16):
    nt, n, _ = adj_tensor.shape
    d = h.shape[1]
    nb = n // bm
    t_fwd = nt // 2

    full = lambda shape: pl.BlockSpec(shape, lambda i, t: (0,) * len(shape))
    out_shapes = [jax.ShapeDtypeStruct((n, d), jnp.float32)]
    out_specs = [pl.BlockSpec((bm, d), lambda i, t: (i, 0))]
    if write_bf16:
        out_shapes.append(jax.ShapeDtypeStruct((nt, n, n), jnp.bfloat16))
        out_specs.append(pl.BlockSpec((1, bm, n), lambda i, t: (t, i, 0)))
    grid_spec = pltpu.PrefetchScalarGridSpec(
        num_scalar_prefetch=0,
        grid=(nb, nt),
        in_specs=[
            pl.BlockSpec((1, bm, n), lambda i, t: (t, i, 0)),
            full((n, d)),
            full((nt, d, d)),
            full((nt, d)),
            full((3 * d, d)),
            full((3 * d, d)),
            full((3 * d, d)),
            full((1, d)),
            full((1, d)),
            full((1, d)),
        ],
        out_specs=out_specs,
        scratch_shapes=[
            pltpu.VMEM((nt, n, d), jnp.bfloat16),
            pltpu.VMEM((bm, d), jnp.float32),
            pltpu.VMEM((bm, d), jnp.float32),
        ],
    )

    def kern(*refs):
        ins = refs[:10]
        if write_bf16:
            out, adj16 = refs[10], refs[11]
            scratch = refs[12:]
        else:
            out, adj16 = refs[10], None
            scratch = refs[11:]
        _step_kernel(*ins, out, adj16, *scratch,
                     t_fwd=t_fwd, bm=bm, write_bf16=write_bf16)

    res = pl.pallas_call(
        kern,
        grid_spec=grid_spec,
        out_shape=out_shapes,
        compiler_params=pltpu.CompilerParams(
            dimension_semantics=("arbitrary", "arbitrary"),
        ),
    )(adj_tensor, h, W_edge, b_edge, Wz, Wr, Wh,
      bz.reshape(1, d), br.reshape(1, d), bh.reshape(1, d))
    return res if write_bf16 else (res[0], None)


def kernel(adj_tensor, node_labels, n_prop_steps, emb, W_edge, b_edge,
           Wz, bz, Wr, br, Wh, bh):
    h0 = _embedding_gather(node_labels, emb)

    h1, adj16 = _prop_step(adj_tensor, h0, W_edge, b_edge, Wz, bz, Wr, br,
                           Wh, bh, bm=256, write_bf16=True)

    def body(_, h):
        h2, _ = _prop_step(adj16, h, W_edge, b_edge, Wz, bz, Wr, br,
                           Wh, bh, bm=256, write_bf16=False)
        return h2

    return jax.lax.fori_loop(0, n_prop_steps - 1, body, h1)


# speedup vs baseline: 1.5986x; 1.5212x over previous
import functools

import jax
import jax.numpy as jnp
from jax.experimental import pallas as pl
from jax.experimental.pallas import tpu as pltpu


def _gather_kernel(labels_ref, emb_ref, out_ref):
    n = labels_ref.shape[0]
    lpad = emb_ref.shape[0]
    lab = labels_ref[:]
    iota = jax.lax.broadcasted_iota(jnp.int32, (n, lpad), 1)
    onehot = (lab == iota).astype(jnp.float32)
    out_ref[:] = jnp.dot(onehot, emb_ref[:], preferred_element_type=jnp.float32)


def _embedding_gather(node_labels, emb):
    n = node_labels.shape[0]
    l, d = emb.shape
    lpad = ((l + 127) // 128) * 128
    emb_p = jnp.pad(emb, ((0, lpad - l), (0, 0)))
    labels2d = node_labels.astype(jnp.int32).reshape(n, 1)
    return pl.pallas_call(
        _gather_kernel,
        out_shape=jax.ShapeDtypeStruct((n, d), jnp.float32),
    )(labels2d, emb_p)


def _msgs_kernel(h_ref, wef_ref, bef_ref, out_ref, *, nt, d):
    m = (jnp.dot(h_ref[:], wef_ref[:], preferred_element_type=jnp.float32)
         + bef_ref[:])
    for t in range(nt):
        out_ref[t] = m[:, t * d:(t + 1) * d].astype(jnp.bfloat16)


def _compute_msgs(h, We_flat, be_flat, nt, d):
    n = h.shape[0]
    return pl.pallas_call(
        functools.partial(_msgs_kernel, nt=nt, d=d),
        out_shape=jax.ShapeDtypeStruct((nt, n, d), jnp.bfloat16),
    )(h, We_flat, be_flat)


def _step_kernel(adj_ref, msgs_ref, h_ref, Wz_ref, Wr_ref, Wh_ref,
                 bz_ref, br_ref, bh_ref, *rest, t_fwd, write_bf16):
    if write_bf16:
        out_ref, adj16_ref = rest
    else:
        (out_ref,) = rest
        adj16_ref = None
    nt = msgs_ref.shape[0]
    d = msgs_ref.shape[2]

    slabs = []
    for t in range(nt):
        a = adj_ref[t]
        if a.dtype != jnp.bfloat16:
            a = a.astype(jnp.bfloat16)
        slabs.append(a)
        if write_bf16:
            adj16_ref[t] = a

    a_in = jnp.dot(slabs[0], msgs_ref[0], preferred_element_type=jnp.float32)
    for t in range(1, t_fwd):
        a_in += jnp.dot(slabs[t], msgs_ref[t], preferred_element_type=jnp.float32)
    a_out = jnp.dot(slabs[t_fwd], msgs_ref[t_fwd], preferred_element_type=jnp.float32)
    for t in range(t_fwd + 1, nt):
        a_out += jnp.dot(slabs[t], msgs_ref[t], preferred_element_type=jnp.float32)

    h_blk = h_ref[:]

    def lin3(a, b, c, w_ref, bias_ref):
        return (
            jnp.dot(a, w_ref[0:d, :], preferred_element_type=jnp.float32)
            + jnp.dot(b, w_ref[d:2 * d, :], preferred_element_type=jnp.float32)
            + jnp.dot(c, w_ref[2 * d:3 * d, :], preferred_element_type=jnp.float32)
            + bias_ref[:]
        )

    z = jax.nn.sigmoid(lin3(a_in, a_out, h_blk, Wz_ref, bz_ref))
    r = jax.nn.sigmoid(lin3(a_in, a_out, h_blk, Wr_ref, br_ref))
    h_hat = jnp.tanh(lin3(a_in, a_out, r * h_blk, Wh_ref, bh_ref))
    out_ref[:] = (1.0 - z) * h_blk + z * h_hat


def _prop_step(adj_tensor, msgs, h, Wz, bz, Wr, br, Wh, bh,
               *, bm, write_bf16):
    nt, n, _ = adj_tensor.shape
    d = h.shape[1]
    nb = n // bm
    t_fwd = nt // 2

    full = lambda shape: pl.BlockSpec(shape, lambda i: (0,) * len(shape))
    out_shapes = [jax.ShapeDtypeStruct((n, d), jnp.float32)]
    out_specs = [pl.BlockSpec((bm, d), lambda i: (i, 0))]
    if write_bf16:
        out_shapes.append(jax.ShapeDtypeStruct((nt, n, n), jnp.bfloat16))
        out_specs.append(pl.BlockSpec((nt, bm, n), lambda i: (0, i, 0)))

    res = pl.pallas_call(
        functools.partial(_step_kernel, t_fwd=t_fwd, write_bf16=write_bf16),
        grid=(nb,),
        in_specs=[
            pl.BlockSpec((nt, bm, n), lambda i: (0, i, 0)),
            full((nt, n, d)),
            pl.BlockSpec((bm, d), lambda i: (i, 0)),
            full((3 * d, d)),
            full((3 * d, d)),
            full((3 * d, d)),
            full((1, d)),
            full((1, d)),
            full((1, d)),
        ],
        out_specs=out_specs,
        out_shape=out_shapes,
        compiler_params=pltpu.CompilerParams(
            dimension_semantics=("arbitrary",),
        ),
    )(adj_tensor, msgs, h, Wz, Wr, Wh,
      bz.reshape(1, d), br.reshape(1, d), bh.reshape(1, d))
    return res if write_bf16 else (res[0], None)


def kernel(adj_tensor, node_labels, n_prop_steps, emb, W_edge, b_edge,
           Wz, bz, Wr, br, Wh, bh):
    nt, _, d = W_edge.shape
    We_flat = W_edge.transpose(1, 0, 2).reshape(d, nt * d)
    be_flat = b_edge.reshape(1, nt * d)

    h0 = _embedding_gather(node_labels, emb)

    msgs0 = _compute_msgs(h0, We_flat, be_flat, nt, d)
    h1, adj16 = _prop_step(adj_tensor, msgs0, h0, Wz, bz, Wr, br, Wh, bh,
                           bm=256, write_bf16=True)

    def body(_, h):
        msgs = _compute_msgs(h, We_flat, be_flat, nt, d)
        h2, _ = _prop_step(adj16, msgs, h, Wz, bz, Wr, br, Wh, bh,
                           bm=256, write_bf16=False)
        return h2

    return jax.lax.fori_loop(0, n_prop_steps - 1, body, h1)


# speedup vs baseline: 1.6464x; 1.0299x over previous
import functools

import jax
import jax.numpy as jnp
from jax.experimental import pallas as pl
from jax.experimental.pallas import tpu as pltpu


def _msgs_from_h(h_blk, wef_ref, bef_ref, msgs_out_ref, nt, d):
    m = (jnp.dot(h_blk, wef_ref[:], preferred_element_type=jnp.float32)
         + bef_ref[:])
    for t in range(nt):
        msgs_out_ref[t] = m[:, t * d:(t + 1) * d].astype(jnp.bfloat16)


def _gather_kernel(labels_ref, emb_ref, wef_ref, bef_ref,
                   h_ref, msgs_ref, *, nt, d):
    n = labels_ref.shape[0]
    lpad = emb_ref.shape[0]
    lab = labels_ref[:]
    iota = jax.lax.broadcasted_iota(jnp.int32, (n, lpad), 1)
    onehot = (lab == iota).astype(jnp.float32)
    h0 = jnp.dot(onehot, emb_ref[:], preferred_element_type=jnp.float32)
    h_ref[:] = h0
    _msgs_from_h(h0, wef_ref, bef_ref, msgs_ref, nt, d)


def _gather_and_msgs(node_labels, emb, We_flat, be_flat, nt):
    n = node_labels.shape[0]
    l, d = emb.shape
    lpad = ((l + 127) // 128) * 128
    emb_p = jnp.pad(emb, ((0, lpad - l), (0, 0)))
    labels2d = node_labels.astype(jnp.int32).reshape(n, 1)
    return pl.pallas_call(
        functools.partial(_gather_kernel, nt=nt, d=d),
        out_shape=[
            jax.ShapeDtypeStruct((n, d), jnp.float32),
            jax.ShapeDtypeStruct((nt, n, d), jnp.bfloat16),
        ],
    )(labels2d, emb_p, We_flat, be_flat)


def _step_kernel(adj_ref, msgs_ref, h_ref, Wz_ref, Wr_ref, Wh_ref,
                 bz_ref, br_ref, bh_ref, wef_ref, bef_ref, *rest,
                 t_fwd, write_bf16):
    if write_bf16:
        out_ref, msgs_out_ref, adj16_ref = rest
    else:
        out_ref, msgs_out_ref = rest
        adj16_ref = None
    nt = msgs_ref.shape[0]
    d = msgs_ref.shape[2]

    slabs = []
    for t in range(nt):
        a = adj_ref[t]
        if a.dtype != jnp.bfloat16:
            a = a.astype(jnp.bfloat16)
        slabs.append(a)
        if write_bf16:
            adj16_ref[t] = a

    a_in = jnp.dot(slabs[0], msgs_ref[0], preferred_element_type=jnp.float32)
    for t in range(1, t_fwd):
        a_in += jnp.dot(slabs[t], msgs_ref[t], preferred_element_type=jnp.float32)
    a_out = jnp.dot(slabs[t_fwd], msgs_ref[t_fwd], preferred_element_type=jnp.float32)
    for t in range(t_fwd + 1, nt):
        a_out += jnp.dot(slabs[t], msgs_ref[t], preferred_element_type=jnp.float32)

    h_blk = h_ref[:]

    def lin3(a, b, c, w_ref, bias_ref):
        return (
            jnp.dot(a, w_ref[0:d, :], preferred_element_type=jnp.float32)
            + jnp.dot(b, w_ref[d:2 * d, :], preferred_element_type=jnp.float32)
            + jnp.dot(c, w_ref[2 * d:3 * d, :], preferred_element_type=jnp.float32)
            + bias_ref[:]
        )

    z = jax.nn.sigmoid(lin3(a_in, a_out, h_blk, Wz_ref, bz_ref))
    r = jax.nn.sigmoid(lin3(a_in, a_out, h_blk, Wr_ref, br_ref))
    h_hat = jnp.tanh(lin3(a_in, a_out, r * h_blk, Wh_ref, bh_ref))
    h_new = (1.0 - z) * h_blk + z * h_hat
    out_ref[:] = h_new
    _msgs_from_h(h_new, wef_ref, bef_ref, msgs_out_ref, nt, d)


def _prop_step(adj_tensor, msgs, h, Wz, bz, Wr, br, Wh, bh,
               We_flat, be_flat, *, bm, write_bf16):
    nt, n, _ = adj_tensor.shape
    d = h.shape[1]
    nb = n // bm
    t_fwd = nt // 2

    full = lambda shape: pl.BlockSpec(shape, lambda i: (0,) * len(shape))
    out_shapes = [
        jax.ShapeDtypeStruct((n, d), jnp.float32),
        jax.ShapeDtypeStruct((nt, n, d), jnp.bfloat16),
    ]
    out_specs = [
        pl.BlockSpec((bm, d), lambda i: (i, 0)),
        pl.BlockSpec((nt, bm, d), lambda i: (0, i, 0)),
    ]
    if write_bf16:
        out_shapes.append(jax.ShapeDtypeStruct((nt, n, n), jnp.bfloat16))
        out_specs.append(pl.BlockSpec((nt, bm, n), lambda i: (0, i, 0)))

    return pl.pallas_call(
        functools.partial(_step_kernel, t_fwd=t_fwd, write_bf16=write_bf16),
        grid=(nb,),
        in_specs=[
            pl.BlockSpec((nt, bm, n), lambda i: (0, i, 0)),
            full((nt, n, d)),
            pl.BlockSpec((bm, d), lambda i: (i, 0)),
            full((3 * d, d)),
            full((3 * d, d)),
            full((3 * d, d)),
            full((1, d)),
            full((1, d)),
            full((1, d)),
            full((d, nt * d)),
            full((1, nt * d)),
        ],
        out_specs=out_specs,
        out_shape=out_shapes,
        compiler_params=pltpu.CompilerParams(
            dimension_semantics=("arbitrary",),
        ),
    )(adj_tensor, msgs, h, Wz, Wr, Wh,
      bz.reshape(1, d), br.reshape(1, d), bh.reshape(1, d), We_flat, be_flat)


def kernel(adj_tensor, node_labels, n_prop_steps, emb, W_edge, b_edge,
           Wz, bz, Wr, br, Wh, bh):
    nt, _, d = W_edge.shape
    We_flat = W_edge.transpose(1, 0, 2).reshape(d, nt * d)
    be_flat = b_edge.reshape(1, nt * d)

    h0, msgs0 = _gather_and_msgs(node_labels, emb, We_flat, be_flat, nt)

    h1, msgs1, adj16 = _prop_step(adj_tensor, msgs0, h0, Wz, bz, Wr, br,
                                  Wh, bh, We_flat, be_flat,
                                  bm=256, write_bf16=True)

    def body(_, carry):
        h, msgs = carry
        h2, msgs2 = _prop_step(adj16, msgs, h, Wz, bz, Wr, br, Wh, bh,
                               We_flat, be_flat, bm=256, write_bf16=False)
        return (h2, msgs2)

    h_fin, _ = jax.lax.fori_loop(0, n_prop_steps - 1, body, (h1, msgs1))
    return h_fin
